# baseline (speedup 1.0000x reference)
import jax
import jax.numpy as jnp
from jax import lax
from jax.experimental import pallas as pl
from jax.experimental.pallas import tpu as pltpu


def _make_kernel(N, BK, KSTEPS):
    KPAD = KSTEPS * BK
    KTAIL = N - (KSTEPS - 1) * BK

    def _gcn_kernel(x_ref, w_ref, b_ref, adj_ref, out_ref, support_ref, acc_ref):
        k = pl.program_id(1)

        @pl.when((pl.program_id(0) == 0) & (k == 0))
        def _():
            support_ref[pl.ds(0, N), :] = jnp.dot(
                x_ref[...], w_ref[...], preferred_element_type=jnp.float32
            )
            support_ref[pl.ds(N, KPAD - N), :] = jnp.zeros(
                (KPAD - N, support_ref.shape[1]), jnp.float32
            )

        @pl.when(k == 0)
        def _():
            acc_ref[...] = jnp.zeros_like(acc_ref)

        sup = support_ref[pl.ds(k * BK, BK), :]

        @pl.when(k < KSTEPS - 1)
        def _():
            acc_ref[...] += jnp.dot(
                adj_ref[...], sup, preferred_element_type=jnp.float32
            )

        @pl.when(k == KSTEPS - 1)
        def _():
            col = lax.broadcasted_iota(jnp.int32, adj_ref.shape, 1)
            a = jnp.where(col < KTAIL, adj_ref[...], 0.0)
            acc = acc_ref[...] + jnp.dot(a, sup, preferred_element_type=jnp.float32)
            h = jnp.maximum(acc + b_ref[...], 0.0)
            m = jnp.max(h, axis=1, keepdims=True)
            s = h - m
            lse = jnp.log(jnp.sum(jnp.exp(s), axis=1, keepdims=True))
            out_ref[...] = s - lse

    return _gcn_kernel


def kernel(x, adj, W, b):
    N, nfeat = x.shape
    nhid = W.shape[1]
    BR = 2000
    BK = 2048
    KSTEPS = pl.cdiv(N, BK)

    return pl.pallas_call(
        _make_kernel(N, BK, KSTEPS),
        grid=(pl.cdiv(N, BR), KSTEPS),
        in_specs=[
            pl.BlockSpec((N, nfeat), lambda r, k: (0, 0)),
            pl.BlockSpec((nfeat, nhid), lambda r, k: (0, 0)),
            pl.BlockSpec((1, nhid), lambda r, k: (0, 0)),
            pl.BlockSpec((BR, BK), lambda r, k: (r, k)),
        ],
        out_specs=pl.BlockSpec((BR, nhid), lambda r, k: (r, 0)),
        out_shape=jax.ShapeDtypeStruct((N, nhid), jnp.float32),
        scratch_shapes=[
            pltpu.VMEM((KSTEPS * BK, nhid), jnp.float32),
            pltpu.VMEM((BR, nhid), jnp.float32),
        ],
        compiler_params=pltpu.CompilerParams(
            vmem_limit_bytes=100 * 1024 * 1024,
        ),
    )(x, W, b.reshape(1, nhid), adj)
